# baseline (speedup 1.0000x reference)
import functools

import jax
import jax.numpy as jnp
from jax import lax
from jax.experimental import pallas as pl
from jax.experimental.pallas import tpu as pltpu
from jax.experimental.pallas import tpu_sc as plsc

NUM_ROWS = 16
DIM = 128
NUM_IDX = 16384

B_SC = 8192
B_TC = NUM_IDX - B_SC

NC = 2
NS = 16
NW = NC * NS
B_PER_W = B_SC // NW
CHUNK = 128
NCHUNK = B_PER_W // CHUNK

_mesh = plsc.VectorSubcoreMesh(core_axis_name="c", subcore_axis_name="s")


@functools.partial(
    pl.kernel,
    mesh=_mesh,
    out_type=jax.ShapeDtypeStruct((B_SC, DIM), jnp.float32),
    scratch_types=[
        pltpu.VMEM((NCHUNK, CHUNK), jnp.int32),
        pltpu.VMEM((B_PER_W, DIM), jnp.float32),
        pltpu.VMEM_SHARED((NUM_ROWS, DIM), jnp.float32),
        pltpu.SemaphoreType.DMA,
        pltpu.SemaphoreType.DMA,
    ],
)
def _sc_gather(table_hbm, idx_hbm, out_hbm, idx_v, rows_v, table_sh,
               gsem, wsem):
    cid = lax.axis_index("c")
    sid = lax.axis_index("s")
    wid = sid * NC + cid
    base = wid * B_PER_W

    @pl.when(sid == 0)
    def _():
        pltpu.sync_copy(table_hbm, table_sh)

    pltpu.sync_copy(idx_hbm.at[wid], idx_v)
    plsc.subcore_barrier()

    gathers = []
    for j in range(NCHUNK):
        gathers.append(
            pltpu.async_copy(
                table_sh.at[idx_v.at[j]],
                rows_v.at[pl.ds(j * CHUNK, CHUNK), :],
                gsem,
            )
        )
    writes = []
    for j in range(NCHUNK):
        gathers[j].wait()
        writes.append(
            pltpu.async_copy(
                rows_v.at[pl.ds(j * CHUNK, CHUNK), :],
                out_hbm.at[pl.ds(base + j * CHUNK, CHUNK), :],
                wsem,
            )
        )
    for w in writes:
        w.wait()


TC_BLOCK = 1024


def _tc_body(idx_ref, table_ref, out_ref):
    idx = idx_ref[:]
    onehot = (idx[:, None]
              == lax.broadcasted_iota(jnp.int32, (TC_BLOCK, NUM_ROWS), 1))
    out_ref[:, :] = jnp.dot(onehot.astype(jnp.float32), table_ref[:, :],
                            preferred_element_type=jnp.float32)


_tc_lookup = pl.pallas_call(
    _tc_body,
    grid=(B_TC // TC_BLOCK,),
    in_specs=[
        pl.BlockSpec((TC_BLOCK,), lambda i: (i,)),
        pl.BlockSpec((NUM_ROWS, DIM), lambda i: (0, 0)),
    ],
    out_specs=pl.BlockSpec((TC_BLOCK, DIM), lambda i: (i, 0)),
    out_shape=jax.ShapeDtypeStruct((B_TC, DIM), jnp.float32),
)


def kernel(emotions, emb_table):
    idx = emotions.astype(jnp.int32)
    sc_out = _sc_gather(emb_table, idx[:B_SC].reshape(NW, NCHUNK, CHUNK))
    tc_out = _tc_lookup(idx[B_SC:], emb_table)
    return jnp.concatenate([sc_out, tc_out], axis=0)


# speedup vs baseline: 1.2080x; 1.2080x over previous
import functools

import jax
import jax.numpy as jnp
from jax import lax
from jax.experimental import pallas as pl
from jax.experimental.pallas import tpu as pltpu
from jax.experimental.pallas import tpu_sc as plsc

NUM_ROWS = 16
DIM = 128
NUM_IDX = 16384

NC = 2
NS = 16
NW = NC * NS
B_PER_W = NUM_IDX // NW
IDX_ROWS = 4
CHUNKS = (32, 96, 128, 128, 128)

_mesh = plsc.VectorSubcoreMesh(core_axis_name="c", subcore_axis_name="s")


@functools.partial(
    pl.kernel,
    mesh=_mesh,
    out_type=jax.ShapeDtypeStruct((NUM_IDX, DIM), jnp.float32),
    scratch_types=[
        pltpu.VMEM((IDX_ROWS, 128), jnp.int32),
        pltpu.VMEM((B_PER_W, DIM), jnp.float32),
        pltpu.VMEM_SHARED((NUM_ROWS, DIM), jnp.float32),
        pltpu.SemaphoreType.DMA,
        pltpu.SemaphoreType.DMA,
    ],
)
def _gather_kernel(table_hbm, idx_hbm, out_hbm, idx_v, rows_v, table_sh,
                   gsem, wsem):
    cid = lax.axis_index("c")
    sid = lax.axis_index("s")
    wid = sid * NC + cid
    base = wid * B_PER_W

    @pl.when(sid == 0)
    def _():
        pltpu.sync_copy(table_hbm, table_sh)

    pltpu.sync_copy(idx_hbm.at[wid], idx_v)
    plsc.subcore_barrier()

    gathers = []
    off = 0
    for sz in CHUNKS:
        r, c = divmod(off, 128)
        idx_slice = idx_v.at[r] if sz == 128 else idx_v.at[r, pl.ds(c, sz)]
        gathers.append(
            (off, sz,
             pltpu.async_copy(
                 table_sh.at[idx_slice],
                 rows_v.at[pl.ds(off, sz), :],
                 gsem,
             ))
        )
        off += sz
    writes = []
    for off, sz, g in gathers:
        g.wait()
        writes.append(
            pltpu.async_copy(
                rows_v.at[pl.ds(off, sz), :],
                out_hbm.at[pl.ds(base + off, sz), :],
                wsem,
            )
        )
    for w in writes:
        w.wait()


def kernel(emotions, emb_table):
    idx = emotions.astype(jnp.int32).reshape(NW, IDX_ROWS, 128)
    return _gather_kernel(emb_table, idx)


# speedup vs baseline: 1.2194x; 1.0095x over previous
import functools

import jax
import jax.numpy as jnp
from jax import lax
from jax.experimental import pallas as pl
from jax.experimental.pallas import tpu as pltpu
from jax.experimental.pallas import tpu_sc as plsc

NUM_ROWS = 16
DIM = 128
NUM_IDX = 16384

NC = 2
NS = 16
NW = NC * NS
B_PER_W = NUM_IDX // NW
CHUNK = 128
NCHUNK = B_PER_W // CHUNK

_mesh = plsc.VectorSubcoreMesh(core_axis_name="c", subcore_axis_name="s")


@functools.partial(
    pl.kernel,
    mesh=_mesh,
    out_type=jax.ShapeDtypeStruct((NUM_IDX, DIM), jnp.float32),
    scratch_types=[
        pltpu.VMEM((NCHUNK, CHUNK), jnp.int32),
        pltpu.VMEM((B_PER_W, DIM), jnp.float32),
        pltpu.VMEM_SHARED((NUM_ROWS, DIM), jnp.float32),
        pltpu.SemaphoreType.DMA,
        pltpu.SemaphoreType.DMA,
    ],
)
def _gather_kernel(table_hbm, idx_hbm, out_hbm, idx_v, rows_v, table_sh,
                   gsem, wsem):
    cid = lax.axis_index("c")
    sid = lax.axis_index("s")
    wid = sid * NC + cid
    base = wid * B_PER_W

    @pl.when(sid == 0)
    def _():
        pltpu.sync_copy(table_hbm, table_sh)

    pltpu.sync_copy(idx_hbm.at[wid], idx_v)
    plsc.subcore_barrier()

    gathers = []
    for j in range(NCHUNK):
        gathers.append(
            pltpu.async_copy(
                table_sh.at[idx_v.at[j]],
                rows_v.at[pl.ds(j * CHUNK, CHUNK), :],
                gsem,
            )
        )
    writes = []
    for j in range(NCHUNK):
        gathers[j].wait()
        writes.append(
            pltpu.async_copy(
                rows_v.at[pl.ds(j * CHUNK, CHUNK), :],
                out_hbm.at[pl.ds(base + j * CHUNK, CHUNK), :],
                wsem,
            )
        )
    for w in writes:
        w.wait()


def kernel(emotions, emb_table):
    idx = emotions.astype(jnp.int32).reshape(NW, NCHUNK, CHUNK)
    return _gather_kernel(emb_table, idx)
